# Initial kernel scaffold; baseline (speedup 1.0000x reference)
import jax
import jax.numpy as jnp
from jax.experimental import pallas as pl


def kernel(labels):
    raise NotImplementedError("write your pallas kernel here")


# speedup vs baseline: 6.1126x; 6.1126x over previous
import numpy as np
import jax
import jax.numpy as jnp
from jax import lax
from jax.experimental import pallas as pl
from jax.experimental.pallas import tpu as pltpu

NUM_VALS = 10
_ROT_A = (13, 15, 26, 6)
_ROT_B = (17, 29, 16, 24)
_LO = np.nextafter(np.float32(-1.0), np.float32(0.0), dtype=np.float32)
_DELTA = np.float32(np.float32(1.0) - _LO)
_SQRT2 = np.float32(np.sqrt(2.0))


def _threefry2x32(k1, k2, x0, x1):
    def rotl(x, d):
        return lax.shift_left(x, np.int32(d)) | lax.shift_right_logical(
            x, np.int32(32 - d)
        )

    def round4(a, b, rots):
        for r in rots:
            a = a + b
            b = rotl(b, r)
            b = b ^ a
        return a, b

    ks0, ks1 = k1, k2
    ks2 = ks0 ^ ks1 ^ np.int32(0x1BD11BDA)
    x0 = x0 + ks0
    x1 = x1 + ks1
    x0, x1 = round4(x0, x1, _ROT_A)
    x0 = x0 + ks1
    x1 = x1 + ks2 + np.int32(1)
    x0, x1 = round4(x0, x1, _ROT_B)
    x0 = x0 + ks2
    x1 = x1 + ks0 + np.int32(2)
    x0, x1 = round4(x0, x1, _ROT_A)
    x0 = x0 + ks0
    x1 = x1 + ks1 + np.int32(3)
    x0, x1 = round4(x0, x1, _ROT_B)
    x0 = x0 + ks1
    x1 = x1 + ks2 + np.int32(4)
    x0, x1 = round4(x0, x1, _ROT_A)
    x0 = x0 + ks2
    x1 = x1 + ks0 + np.int32(5)
    return x0, x1


def _presence_body(lab_ref, out_ref):
    @pl.when(pl.program_id(0) == 0)
    def _init():
        out_ref[...] = jnp.zeros_like(out_ref)

    m = lax.shift_left(jnp.int32(1), lab_ref[...])
    r, c = m.shape
    while r > 8:
        m = m[: r // 2, :] | m[r // 2 :, :]
        r //= 2
    while c > 128:
        m = m[:, : c // 2] | m[:, c // 2 :]
        c //= 2
    out_ref[...] = out_ref[...] | m


def _sample_body(vk1_ref, vk2_ref, vmean_ref, vstd_ref, lab_ref, out_ref):
    blk = lab_ref[...]
    br, bc = blk.shape

    k1 = jnp.full(blk.shape, vk1_ref[0], dtype=jnp.int32)
    k2 = jnp.full(blk.shape, vk2_ref[0], dtype=jnp.int32)
    mean = jnp.full(blk.shape, vmean_ref[0], dtype=jnp.float32)
    std = jnp.full(blk.shape, vstd_ref[0], dtype=jnp.float32)
    for v in range(1, NUM_VALS):
        sel = blk == v
        k1 = jnp.where(sel, vk1_ref[v], k1)
        k2 = jnp.where(sel, vk2_ref[v], k2)
        mean = jnp.where(sel, vmean_ref[v], mean)
        std = jnp.where(sel, vstd_ref[v], std)

    base = pl.program_id(0) * np.int32(br * bc)
    lin = (
        lax.broadcasted_iota(jnp.int32, blk.shape, 0) * np.int32(bc)
        + lax.broadcasted_iota(jnp.int32, blk.shape, 1)
    )
    p = base + lin

    h0, h1 = _threefry2x32(k1, k2, jnp.zeros_like(p), p)
    bits = h0 ^ h1

    fb = lax.shift_right_logical(bits, np.int32(9)) | np.int32(0x3F800000)
    f = lax.bitcast_convert_type(fb, jnp.float32) - np.float32(1.0)
    u = jnp.maximum(_LO, f * _DELTA + _LO)
    nrm = _SQRT2 * lax.erf_inv(u)
    out_ref[...] = mean + std * nrm


def kernel(labels):
    shape = labels.shape
    n_elems = int(np.prod(shape))
    rows = 512
    cols = n_elems // rows
    lab2d = labels.reshape(rows, cols).astype(jnp.int32)

    pres_steps = 4
    pres_br = rows // pres_steps
    ormask = pl.pallas_call(
        _presence_body,
        grid=(pres_steps,),
        in_specs=[pl.BlockSpec((pres_br, cols), lambda i: (i, 0))],
        out_specs=pl.BlockSpec((8, 128), lambda i: (0, 0)),
        out_shape=jax.ShapeDtypeStruct((8, 128), jnp.int32),
    )(lab2d)
    mask = jnp.bitwise_or.reduce(jnp.bitwise_or.reduce(ormask, axis=0))
    present = (lax.shift_right_logical(mask, jnp.arange(NUM_VALS)) & 1).astype(
        jnp.int32
    )

    key = jax.random.key(42)
    k_mean, k_std, k_samp = jax.random.split(key, 3)
    n = jnp.sum(present)
    idx = jnp.cumsum(present) - 1
    means_all = jnp.stack(
        [
            jnp.pad(
                jax.random.uniform(k_mean, (k,), minval=0.0, maxval=255.0),
                (0, NUM_VALS - k),
            )
            for k in range(NUM_VALS + 1)
        ]
    )
    stds_all = jnp.stack(
        [
            jnp.pad(
                jax.random.uniform(k_std, (k,), minval=0.0, maxval=30.0),
                (0, NUM_VALS - k),
            )
            for k in range(NUM_VALS + 1)
        ]
    )
    class_means = means_all[n]
    class_stds = stds_all[n]
    fold_keys = jnp.stack(
        [
            jax.random.key_data(jax.random.fold_in(k_samp, i))
            for i in range(NUM_VALS)
        ]
    ).astype(jnp.int32)

    iv = jnp.clip(idx, 0, NUM_VALS - 1)
    vmean = class_means[iv]
    vstd = class_stds[iv]
    vk1 = fold_keys[iv, 0]
    vk2 = fold_keys[iv, 1]

    steps = 8
    br = rows // steps
    smem = pl.BlockSpec(memory_space=pltpu.SMEM)
    out = pl.pallas_call(
        _sample_body,
        grid=(steps,),
        in_specs=[
            smem,
            smem,
            smem,
            smem,
            pl.BlockSpec((br, cols), lambda i: (i, 0)),
        ],
        out_specs=pl.BlockSpec((br, cols), lambda i: (i, 0)),
        out_shape=jax.ShapeDtypeStruct((rows, cols), jnp.float32),
    )(vk1, vk2, vmean, vstd, lab2d)
    return out.reshape(shape)
